# baseline (speedup 1.0000x reference)
import functools

import jax
import jax.numpy as jnp
from jax.experimental import pallas as pl
from jax.experimental.pallas import tpu as pltpu


def _gate_kernel(n_b, inv_hw, x_hbm, w1_ref, b1_ref, w2_ref, b2_ref, o_hbm,
                 x_buf, o_buf, in_sem, out_sem):
    def dma_in(slot, step):
        pltpu.make_async_copy(x_hbm.at[step], x_buf.at[slot],
                              in_sem.at[slot]).start()

    def wait_in(slot):
        pltpu.make_async_copy(x_buf.at[slot], x_buf.at[slot],
                              in_sem.at[slot]).wait()

    def dma_out(slot, step):
        pltpu.make_async_copy(o_buf.at[slot], o_hbm.at[step],
                              out_sem.at[slot]).start()

    def wait_out(slot):
        pltpu.make_async_copy(o_buf.at[slot], o_buf.at[slot],
                              out_sem.at[slot]).wait()

    dma_in(0, 0)

    def body(b, _):
        cur = jax.lax.rem(b, 2)
        nxt = jax.lax.rem(b + 1, 2)

        @pl.when(b + 1 < n_b)
        def _():
            dma_in(nxt, b + 1)

        wait_in(cur)

        @pl.when(b >= 2)
        def _():
            wait_out(cur)

        x = x_buf[cur]
        s = jnp.sum(x, axis=(1, 2))
        m = jnp.max(x, axis=(1, 2))
        pooled = jnp.stack([s * inv_hw, m], axis=0)
        hidden = jnp.maximum(
            jnp.dot(pooled, w1_ref[...],
                    preferred_element_type=jnp.float32) + b1_ref[...], 0.0)
        att = jnp.dot(hidden, w2_ref[...],
                      preferred_element_type=jnp.float32) + b2_ref[...]
        scale = jax.nn.sigmoid(att[0:1, :] + att[1:2, :])
        o_buf[cur] = x * scale.reshape(x.shape[0], 1, 1)

        dma_out(cur, b)
        return ()

    jax.lax.fori_loop(0, n_b, body, ())
    wait_out(jax.lax.rem(n_b - 2, 2))
    wait_out(jax.lax.rem(n_b - 1, 2))


def kernel(x, w1, b1, w2, b2):
    B, C, H, W = x.shape
    R = w1.shape[1]

    b1r = b1.reshape(1, R)
    b2r = b2.reshape(1, C)

    return pl.pallas_call(
        functools.partial(_gate_kernel, B, 1.0 / float(H * W)),
        out_shape=jax.ShapeDtypeStruct((B, C, H, W), x.dtype),
        in_specs=[pl.BlockSpec(memory_space=pl.ANY),
                  pl.BlockSpec(memory_space=pltpu.VMEM),
                  pl.BlockSpec(memory_space=pltpu.VMEM),
                  pl.BlockSpec(memory_space=pltpu.VMEM),
                  pl.BlockSpec(memory_space=pltpu.VMEM)],
        out_specs=pl.BlockSpec(memory_space=pl.ANY),
        scratch_shapes=[pltpu.VMEM((2, C, H, W), jnp.float32),
                        pltpu.VMEM((2, C, H, W), jnp.float32),
                        pltpu.SemaphoreType.DMA((2,)),
                        pltpu.SemaphoreType.DMA((2,))],
    )(x, w1, b1r, w2, b2r)


# speedup vs baseline: 1.0103x; 1.0103x over previous
import functools

import jax
import jax.numpy as jnp
from jax.experimental import pallas as pl
from jax.experimental.pallas import tpu as pltpu

_DEPTH = 3
_CHUNKS = 4


def _gate_kernel(n_b, inv_hw, x_hbm, w1_ref, b1_ref, w2_ref, b2_ref, o_hbm,
                 x_buf, o_buf, in_sem, out_sem):
    c = x_buf.shape[1]
    ck = c // _CHUNKS

    def dma_in(slot, step):
        for q in range(_CHUNKS):
            pltpu.make_async_copy(x_hbm.at[step, pl.ds(q * ck, ck)],
                                  x_buf.at[slot, pl.ds(q * ck, ck)],
                                  in_sem.at[slot]).start()

    def wait_in(slot):
        for q in range(_CHUNKS):
            pltpu.make_async_copy(x_buf.at[slot, pl.ds(q * ck, ck)],
                                  x_buf.at[slot, pl.ds(q * ck, ck)],
                                  in_sem.at[slot]).wait()

    def dma_out(slot, step):
        for q in range(_CHUNKS):
            pltpu.make_async_copy(o_buf.at[slot, pl.ds(q * ck, ck)],
                                  o_hbm.at[step, pl.ds(q * ck, ck)],
                                  out_sem.at[slot]).start()

    def wait_out(slot):
        for q in range(_CHUNKS):
            pltpu.make_async_copy(o_buf.at[slot, pl.ds(q * ck, ck)],
                                  o_buf.at[slot, pl.ds(q * ck, ck)],
                                  out_sem.at[slot]).wait()

    for p in range(_DEPTH - 1):
        dma_in(p, p)

    def body(b, _):
        cur = jax.lax.rem(b, _DEPTH)

        @pl.when(b + _DEPTH - 1 < n_b)
        def _():
            dma_in(jax.lax.rem(b + _DEPTH - 1, _DEPTH), b + _DEPTH - 1)

        wait_in(cur)

        @pl.when(b >= _DEPTH)
        def _():
            wait_out(cur)

        x = x_buf[cur]
        s = jnp.sum(x, axis=(1, 2))
        m = jnp.max(x, axis=(1, 2))
        pooled = jnp.stack([s * inv_hw, m], axis=0)
        hidden = jnp.maximum(
            jnp.dot(pooled, w1_ref[...],
                    preferred_element_type=jnp.float32) + b1_ref[...], 0.0)
        att = jnp.dot(hidden, w2_ref[...],
                      preferred_element_type=jnp.float32) + b2_ref[...]
        scale = jax.nn.sigmoid(att[0:1, :] + att[1:2, :])
        o_buf[cur] = x * scale.reshape(x.shape[0], 1, 1)

        dma_out(cur, b)
        return ()

    jax.lax.fori_loop(0, n_b, body, ())
    for p in range(_DEPTH):
        wait_out(jax.lax.rem(n_b - _DEPTH + p, _DEPTH))


def kernel(x, w1, b1, w2, b2):
    B, C, H, W = x.shape
    R = w1.shape[1]

    b1r = b1.reshape(1, R)
    b2r = b2.reshape(1, C)

    return pl.pallas_call(
        functools.partial(_gate_kernel, B, 1.0 / float(H * W)),
        out_shape=jax.ShapeDtypeStruct((B, C, H, W), x.dtype),
        in_specs=[pl.BlockSpec(memory_space=pl.ANY),
                  pl.BlockSpec(memory_space=pltpu.VMEM),
                  pl.BlockSpec(memory_space=pltpu.VMEM),
                  pl.BlockSpec(memory_space=pltpu.VMEM),
                  pl.BlockSpec(memory_space=pltpu.VMEM)],
        out_specs=pl.BlockSpec(memory_space=pl.ANY),
        scratch_shapes=[pltpu.VMEM((_DEPTH, C, H, W), jnp.float32),
                        pltpu.VMEM((_DEPTH, C, H, W), jnp.float32),
                        pltpu.SemaphoreType.DMA((_DEPTH,)),
                        pltpu.SemaphoreType.DMA((_DEPTH,))],
    )(x, w1, b1r, w2, b2r)


# speedup vs baseline: 1.3363x; 1.3226x over previous
import functools

import jax
import jax.numpy as jnp
from jax.experimental import pallas as pl
from jax.experimental.pallas import tpu as pltpu


def _gate_kernel(n_batch, inv_hw, x_ref, w1_ref, b1_ref, w2_ref, b2_ref,
                 o_ref):
    x = x_ref[...]
    c = x.shape[0] // n_batch
    s = jnp.sum(x, axis=(1, 2)).reshape(n_batch, c)
    m = jnp.max(x, axis=(1, 2)).reshape(n_batch, c)
    pooled = jnp.concatenate([s * inv_hw, m], axis=0)
    hidden = jnp.maximum(
        jnp.dot(pooled, w1_ref[...],
                preferred_element_type=jnp.float32) + b1_ref[...], 0.0)
    att = jnp.dot(hidden, w2_ref[...],
                  preferred_element_type=jnp.float32) + b2_ref[...]
    scale = jax.nn.sigmoid(att[:n_batch, :] + att[n_batch:, :])
    o_ref[...] = x * scale.reshape(x.shape[0], 1, 1)


def kernel(x, w1, b1, w2, b2):
    B, C, H, W = x.shape
    R = w1.shape[1]

    nb = 2 if B % 2 == 0 else 1
    x3 = x.reshape(B * C, H, W)
    b1r = b1.reshape(1, R)
    b2r = b2.reshape(1, C)

    out = pl.pallas_call(
        functools.partial(_gate_kernel, nb, 1.0 / float(H * W)),
        out_shape=jax.ShapeDtypeStruct((B * C, H, W), x.dtype),
        grid=(B // nb,),
        in_specs=[pl.BlockSpec((nb * C, H, W), lambda b: (b, 0, 0)),
                  pl.BlockSpec((C, R), lambda b: (0, 0)),
                  pl.BlockSpec((1, R), lambda b: (0, 0)),
                  pl.BlockSpec((R, C), lambda b: (0, 0)),
                  pl.BlockSpec((1, C), lambda b: (0, 0))],
        out_specs=pl.BlockSpec((nb * C, H, W), lambda b: (b, 0, 0)),
        compiler_params=pltpu.CompilerParams(
            dimension_semantics=("parallel",),
            vmem_limit_bytes=128 * 1024 * 1024),
    )(x3, w1, b1r, w2, b2r)

    return out.reshape(B, C, H, W)
